# baseline (speedup 1.0000x reference)
import functools

import jax
import jax.numpy as jnp
from jax import lax
from jax.experimental import pallas as pl
from jax.experimental.pallas import tpu as pltpu
from jax.experimental.pallas import tpu_sc as plsc

S, N, D = 16, 8192, 3
QT = 512
HALF = N // 2
FLAT = N * D


NG = QT // 128


def _corr_body(mvt_ref, pts_ref, corr_ref):
    qxb = [jnp.broadcast_to(mvt_ref[0:1, g * 128:(g + 1) * 128], (8, 128))
           for g in range(NG)]
    qyb = [jnp.broadcast_to(mvt_ref[1:2, g * 128:(g + 1) * 128], (8, 128))
           for g in range(NG)]
    qzb = [jnp.broadcast_to(mvt_ref[2:3, g * 128:(g + 1) * 128], (8, 128))
           for g in range(NG)]
    row = lax.broadcasted_iota(jnp.int32, (8, 128), 0)
    big = jnp.full((8, 128), jnp.float32(jnp.inf))
    zero = jnp.zeros((8, 128), jnp.int32)

    def chunk(c, carry):
        rmins, ridxs = carry
        ptc = pts_ref[0, pl.ds(c * 8, 8), :]
        px = ptc[:, 0:1]
        py = ptc[:, 1:2]
        pz = ptc[:, 2:3]
        jc = row + c * 8
        new_m, new_i = [], []
        for g in range(NG):
            dx = px - qxb[g]
            dy = py - qyb[g]
            dz = pz - qzb[g]
            d8 = dx * dx + dy * dy + dz * dz
            take = d8 < rmins[g]
            new_m.append(jnp.minimum(rmins[g], d8))
            new_i.append(jnp.where(take, jc, ridxs[g]))
        return tuple(new_m), tuple(new_i)

    rmins, ridxs = lax.fori_loop(
        0, N // 8, chunk,
        (tuple(big for _ in range(NG)), tuple(zero for _ in range(NG))),
        unroll=2)
    for g in range(NG):
        m = jnp.min(rmins[g], axis=0, keepdims=True)
        cand = jnp.where(rmins[g] == m, ridxs[g], N)
        corr_ref[0, 0, pl.ds(g * 128, 128)] = jnp.min(cand, axis=0)


def _correspond(mvt, ximcp):
    return pl.pallas_call(
        _corr_body,
        grid=(S, N // QT),
        in_specs=[
            pl.BlockSpec((D, QT), lambda s, q: (0, q)),
            pl.BlockSpec((1, N, D), lambda s, q: (s, 0, 0)),
        ],
        out_specs=pl.BlockSpec((1, 1, QT), lambda s, q: (s, 0, q)),
        out_shape=jax.ShapeDtypeStruct((S, 1, N), jnp.int32),
    )(mvt, ximcp)


def _gather(xrflat, corr):
    mesh = plsc.VectorSubcoreMesh(core_axis_name="c", subcore_axis_name="s")

    @functools.partial(
        pl.kernel,
        mesh=mesh,
        out_type=jax.ShapeDtypeStruct((S * FLAT,), jnp.float32),
        compiler_params=pltpu.CompilerParams(needs_layout_passes=False),
        scratch_types=[
            pltpu.VMEM((FLAT,), jnp.float32),
            pltpu.VMEM((HALF,), jnp.int32),
            pltpu.VMEM((D * HALF,), jnp.float32),
        ],
    )
    def k(xr_hbm, corr_hbm, out_hbm, table_v, corr_v, out_v):
        wid = lax.axis_index("s") * 2 + lax.axis_index("c")
        sidx = wid // 2
        h = wid % 2
        pltpu.sync_copy(xr_hbm.at[pl.ds(sidx * FLAT, FLAT)], table_v)
        pltpu.sync_copy(corr_hbm.at[pl.ds(sidx * N + h * HALF, HALF)], corr_v)
        @plsc.parallel_loop(0, (D * HALF) // 16, 1)
        def body(g):
            kv = g * 16 + lax.iota(jnp.int32, 16)
            iv = kv // 3
            rv = kv - iv * 3
            cv = plsc.load_gather(corr_v, [iv])
            vals = plsc.load_gather(table_v, [cv * 3 + rv])
            out_v[pl.ds(g * 16, 16)] = vals
        pltpu.sync_copy(out_v,
                        out_hbm.at[pl.ds(sidx * FLAT + h * D * HALF, D * HALF)])

    return k(xrflat.reshape(S * FLAT), corr.reshape(S * N))


def kernel(x):
    xc = x - x.mean(axis=1, keepdims=True)
    scale = jnp.sqrt(jnp.sum(xc * xc, axis=(1, 2), keepdims=True)) + 1e-12
    x_n = xc / scale
    cov = jnp.einsum('snd,sne->sde', x_n, x_n) / N
    _, v = jnp.linalg.eigh(cov)
    x_imcp = jnp.einsum('snd,sde->sne', x_n, v)
    mv = x_imcp.mean(axis=0)
    xr = jnp.einsum('snd,sde->sne', xc, v)

    corr = _correspond(mv.T, x_imcp).reshape(S, N)

    flat = _gather(xr.reshape(S, FLAT), corr).reshape(S, FLAT)

    mean_shape = flat.mean(axis=0)
    xcd = flat - mean_shape[None, :]
    gram = (xcd @ xcd.T) / (S - 1)
    evals_a, evecs_a = jnp.linalg.eigh(gram)
    evals = evals_a[::-1]
    evecs = evecs_a[:, ::-1]
    comps = (xcd.T @ evecs) / (jnp.sqrt(jnp.maximum(evals, 1e-12) * (S - 1))[None, :])
    return mean_shape, evals, comps.T
